# baseline (speedup 1.0000x reference)
import functools

import jax
import jax.numpy as jnp
from jax.experimental import pallas as pl
from jax.experimental.pallas import tpu as pltpu

EMBED = 64
NUM_ROWS = 5


def _tc_body(city_ref, table_ref, out_ref):
    c = city_ref[...]
    r = c.shape[0]
    cb = jnp.broadcast_to(c, (r, 8))
    iota8 = jax.lax.broadcasted_iota(jnp.int32, (r, 8), 1)
    onehot = (cb == iota8).astype(jnp.float32)
    out_ref[...] = jax.lax.dot_general(
        onehot, table_ref[...],
        dimension_numbers=(((1,), (0,)), ((), ())),
        preferred_element_type=jnp.float32)


@functools.partial(jax.jit, static_argnames=("block_r",))
def _tc_embed(table, idx2, block_r):
    b = idx2.shape[0]
    nb = b // block_r
    table8 = jnp.zeros((8, EMBED), jnp.float32).at[:NUM_ROWS].set(table)
    return pl.pallas_call(
        _tc_body,
        grid=(nb,),
        in_specs=[
            pl.BlockSpec((block_r, 1), lambda i: (i, 0)),
            pl.BlockSpec((8, EMBED), lambda i: (0, 0)),
        ],
        out_specs=pl.BlockSpec((block_r, EMBED), lambda i: (i, 0)),
        out_shape=jax.ShapeDtypeStruct((b, EMBED), jnp.float32),
        compiler_params=pltpu.CompilerParams(
            dimension_semantics=("arbitrary",)),
    )(idx2, table8)


def kernel(city, table):
    b0, b1 = city.shape
    bflat = b0 * b1
    idx2 = city.reshape(bflat, 1)
    out = _tc_embed(table, idx2, 4096)
    return out.reshape(b0, b1, EMBED)


# speedup vs baseline: 1.1495x; 1.1495x over previous
import functools

import jax
import jax.numpy as jnp
from jax import lax
from jax.experimental import pallas as pl
from jax.experimental.pallas import tpu as pltpu
from jax.experimental.pallas import tpu_sc as plsc

EMBED = 64
NUM_ROWS = 5
SEQ = 200


@functools.partial(jax.jit, static_argnames=("rows_pc",))
def _sc_embed(table, idx_flat, rows_pc):
    info = plsc.get_sparse_core_info()
    nc, ns = info.num_cores, info.num_subcores
    nw = nc * ns
    b = idx_flat.shape[0]
    n_rows_total = b // SEQ
    rows_per_w = n_rows_total // nw
    chunk = rows_pc * SEQ
    n_chunks = rows_per_w // rows_pc
    n_pairs = n_chunks // 2
    assert rows_per_w % (2 * rows_pc) == 0
    b_per_w = b // nw

    mesh = plsc.VectorSubcoreMesh(core_axis_name="c", subcore_axis_name="s")

    @functools.partial(
        pl.kernel,
        mesh=mesh,
        compiler_params=pltpu.CompilerParams(
            use_tc_tiling_on_sc=False, needs_layout_passes=False),
        out_type=jax.ShapeDtypeStruct((n_rows_total, SEQ, EMBED), jnp.float32),
        scratch_types=[
            pltpu.VMEM((2, chunk), jnp.int32),
            pltpu.VMEM((2, chunk, EMBED), jnp.float32),
            pltpu.VMEM_SHARED((NUM_ROWS, EMBED), jnp.float32),
            pltpu.SemaphoreType.DMA,
            pltpu.SemaphoreType.DMA,
            pltpu.SemaphoreType.DMA,
            pltpu.SemaphoreType.DMA,
            pltpu.SemaphoreType.DMA,
            pltpu.SemaphoreType.DMA,
        ],
    )
    def body(table_hbm, idx_hbm, out_hbm, idx_v, rows_v, table_sh,
             si0, si1, sg0, sg1, so0, so1):
        sem_idx = (si0, si1)
        sem_g = (sg0, sg1)
        sem_out = (so0, so1)
        wid = lax.axis_index("s") * nc + lax.axis_index("c")
        base = wid * b_per_w
        rbase = wid * rows_per_w

        @pl.when(lax.axis_index("s") == 0)
        def _():
            pltpu.sync_copy(table_hbm, table_sh)

        plsc.subcore_barrier()

        for slot in range(2):
            pltpu.async_copy(
                idx_hbm.at[pl.ds(base + slot * chunk, chunk)],
                idx_v.at[slot], sem_idx[slot])

        def pair_body(g, carry):
            for slot in range(2):
                i = 2 * g + slot
                off = base + i * chunk
                r0 = rbase + i * rows_pc

                @pl.when(g > 0)
                def _():
                    for jj in range(rows_pc):
                        pltpu.make_async_copy(
                            rows_v.at[slot, pl.ds(jj * SEQ, SEQ)],
                            out_hbm.at[r0 - 2 * rows_pc + jj],
                            sem_out[slot]).wait()

                pltpu.make_async_copy(
                    idx_hbm.at[pl.ds(off, chunk)],
                    idx_v.at[slot], sem_idx[slot]).wait()

                pltpu.async_copy(
                    table_sh.at[idx_v.at[slot]],
                    rows_v.at[slot], sem_g[slot]).wait()

                for jj in range(rows_pc):
                    pltpu.async_copy(
                        rows_v.at[slot, pl.ds(jj * SEQ, SEQ)],
                        out_hbm.at[r0 + jj], sem_out[slot])

                @pl.when(i + 2 < n_chunks)
                def _():
                    pltpu.async_copy(
                        idx_hbm.at[pl.ds(off + 2 * chunk, chunk)],
                        idx_v.at[slot], sem_idx[slot])
            return carry

        lax.fori_loop(0, n_pairs, pair_body, 0)

        for slot in range(2):
            i = 2 * (n_pairs - 1) + slot
            for jj in range(rows_pc):
                pltpu.make_async_copy(
                    rows_v.at[slot, pl.ds(jj * SEQ, SEQ)],
                    out_hbm.at[rbase + i * rows_pc + jj],
                    sem_out[slot]).wait()

    return body(table, idx_flat)


def kernel(city, table):
    b0, b1 = city.shape
    idx_flat = city.reshape(b0 * b1)
    return _sc_embed(table, idx_flat, 4)


# speedup vs baseline: 1.2074x; 1.0504x over previous
import functools

import jax
import jax.numpy as jnp
from jax import lax
from jax.experimental import pallas as pl
from jax.experimental.pallas import tpu as pltpu
from jax.experimental.pallas import tpu_sc as plsc

EMBED = 64
NUM_ROWS = 5
PACK = 4
QROWS = NUM_ROWS ** PACK
QEMBED = EMBED * PACK


@functools.partial(jax.jit, static_argnames=("chunk_q",))
def _sc_embed(qtable, idx_flat, chunk_q):
    info = plsc.get_sparse_core_info()
    nc, ns = info.num_cores, info.num_subcores
    nw = nc * ns
    b = idx_flat.shape[0]
    bq = b // PACK
    chunk = chunk_q * PACK
    assert bq % (nw * chunk_q * 2) == 0
    b_per_w = b // nw
    bq_per_w = bq // nw
    n_chunks = bq_per_w // chunk_q
    n_pairs = n_chunks // 2

    mesh = plsc.VectorSubcoreMesh(core_axis_name="c", subcore_axis_name="s")

    @functools.partial(
        pl.kernel,
        mesh=mesh,
        compiler_params=pltpu.CompilerParams(
            use_tc_tiling_on_sc=False, needs_layout_passes=False),
        out_type=jax.ShapeDtypeStruct((bq, QEMBED), jnp.float32),
        scratch_types=[
            pltpu.VMEM((2, chunk), jnp.int32),
            pltpu.VMEM((2, chunk_q), jnp.int32),
            pltpu.VMEM((2, chunk_q, QEMBED), jnp.float32),
            pltpu.VMEM_SHARED((QROWS, QEMBED), jnp.float32),
            pltpu.SemaphoreType.DMA,
            pltpu.SemaphoreType.DMA,
            pltpu.SemaphoreType.DMA,
            pltpu.SemaphoreType.DMA,
            pltpu.SemaphoreType.DMA,
            pltpu.SemaphoreType.DMA,
        ],
    )
    def body(qtable_hbm, idx_hbm, out_hbm, idx_raw, idx_q, rows_v, qtable_sh,
             si0, si1, sg0, sg1, so0, so1):
        sem_idx = (si0, si1)
        sem_g = (sg0, sg1)
        sem_out = (so0, so1)
        wid = lax.axis_index("s") * nc + lax.axis_index("c")
        base = wid * b_per_w
        qbase = wid * bq_per_w

        @pl.when(lax.axis_index("s") == 0)
        def _():
            pltpu.sync_copy(qtable_hbm, qtable_sh)

        plsc.subcore_barrier()

        iota4 = lax.iota(jnp.int32, 16) * PACK

        for slot in range(2):
            pltpu.async_copy(
                idx_hbm.at[pl.ds(base + slot * chunk, chunk)],
                idx_raw.at[slot], sem_idx[slot])

        def pair_body(g, carry):
            for slot in range(2):
                i = 2 * g + slot
                off = base + i * chunk
                qoff = qbase + i * chunk_q

                @pl.when(g > 0)
                def _():
                    pltpu.make_async_copy(
                        rows_v.at[slot],
                        out_hbm.at[pl.ds(qoff - 2 * chunk_q, chunk_q)],
                        sem_out[slot]).wait()

                pltpu.make_async_copy(
                    idx_hbm.at[pl.ds(off, chunk)],
                    idx_raw.at[slot], sem_idx[slot]).wait()

                for j in range(chunk_q // 16):
                    g0 = plsc.load_gather(idx_raw.at[slot], [iota4 + j * 64])
                    g1 = plsc.load_gather(idx_raw.at[slot], [iota4 + (j * 64 + 1)])
                    g2 = plsc.load_gather(idx_raw.at[slot], [iota4 + (j * 64 + 2)])
                    g3 = plsc.load_gather(idx_raw.at[slot], [iota4 + (j * 64 + 3)])
                    code = ((g0 * NUM_ROWS + g1) * NUM_ROWS + g2) * NUM_ROWS + g3
                    idx_q[slot, pl.ds(j * 16, 16)] = code

                pltpu.async_copy(
                    qtable_sh.at[idx_q.at[slot]],
                    rows_v.at[slot], sem_g[slot]).wait()

                pltpu.async_copy(
                    rows_v.at[slot],
                    out_hbm.at[pl.ds(qoff, chunk_q)], sem_out[slot])

                @pl.when(i + 2 < n_chunks)
                def _():
                    pltpu.async_copy(
                        idx_hbm.at[pl.ds(off + 2 * chunk, chunk)],
                        idx_raw.at[slot], sem_idx[slot])
            return carry

        lax.fori_loop(0, n_pairs, pair_body, 0)

        for slot in range(2):
            i = 2 * (n_pairs - 1) + slot
            pltpu.make_async_copy(
                rows_v.at[slot],
                out_hbm.at[pl.ds(qbase + i * chunk_q, chunk_q)],
                sem_out[slot]).wait()

    return body(qtable, idx_flat)


def kernel(city, table):
    b0, b1 = city.shape
    idx_flat = city.reshape(b0 * b1)
    t = table
    s5 = (NUM_ROWS,) * PACK + (EMBED,)
    qtable = jnp.concatenate([
        jnp.broadcast_to(t[:, None, None, None, :], s5),
        jnp.broadcast_to(t[None, :, None, None, :], s5),
        jnp.broadcast_to(t[None, None, :, None, :], s5),
        jnp.broadcast_to(t[None, None, None, :, :], s5),
    ], axis=-1).reshape(QROWS, QEMBED)
    out = _sc_embed(qtable, idx_flat, 128)
    return out.reshape(b0, b1, EMBED)
